# Initial kernel scaffold; baseline (speedup 1.0000x reference)
import jax
import jax.numpy as jnp
from jax.experimental import pallas as pl


def kernel(x, edge_index, enc_W, enc_b, ln_w, ln_b, t, W1, b1, mlp_ln_w, mlp_ln_b, W2, b2, lin_W, lin_b, out_W, out_b):
    raise NotImplementedError("write your pallas kernel here")


# speedup vs baseline: 9.1642x; 9.1642x over previous
import functools

import jax
import jax.numpy as jnp
from jax import lax
from jax.experimental import pallas as pl
from jax.experimental.pallas import tpu as pltpu
from jax.experimental.pallas import tpu_sc as plsc

_N = 10000
_E = 320000
_D = 128
_H = 256
_L = 3

_C = 128
_NCH = _E // _C
_NSUB = 16
_NCORE = 2
_ZR = 208
_RPS = 624
_REM = _N - _RPS * _NSUB


def _graph_ln(h, w, b, eps=1e-5):
    mu = jnp.mean(h)
    var = jnp.mean((h - mu) ** 2)
    return (h - mu) / (jnp.sqrt(var) + eps) * w + b


def _encode_body(x_ref, w_ref, b_ref, o_ref):
    o_ref[...] = (
        jnp.dot(x_ref[...], w_ref[...], preferred_element_type=jnp.float32)
        + b_ref[...]
    )


def _pre_body(h_ref, lnw_ref, lnb_ref, t_ref, z_ref, tab_ref):
    h = h_ref[...]
    z = jnp.maximum(_graph_ln(h, lnw_ref[...], lnb_ref[...]), 0.0)
    z_ref[...] = z
    msg = z + 1e-7
    a = jnp.exp(msg * t_ref[0, 0])
    tab_ref[0:_N, :] = a
    tab_ref[_N : 2 * _N, :] = msg * a


def _post_body(h_ref, z_ref, s_ref, w1_ref, b1_ref, lnw_ref, lnb_ref,
               w2_ref, b2_ref, o_ref):
    s1 = s_ref[0:_N, :]
    s2 = s_ref[_N : 2 * _N, :]
    out = s2 / (s1 + 1e-16) + z_ref[...]
    h1 = (
        jnp.dot(out, w1_ref[...], preferred_element_type=jnp.float32)
        + b1_ref[...]
    )
    g = jnp.maximum(_graph_ln(h1, lnw_ref[...], lnb_ref[...]), 0.0)
    o_ref[...] = (
        h_ref[...]
        + jnp.dot(g, w2_ref[...], preferred_element_type=jnp.float32)
        + b2_ref[...]
    )


def _head_body(h_ref, lw_ref, lb_ref, ow_ref, ob_ref, o_ref):
    g = jnp.maximum(
        jnp.dot(h_ref[...], lw_ref[...], preferred_element_type=jnp.float32)
        + lb_ref[...],
        0.0,
    )
    o_ref[...] = (
        jnp.dot(g, ow_ref[...], preferred_element_type=jnp.float32)
        + ob_ref[...]
    )


_encode = pl.pallas_call(
    _encode_body, out_shape=jax.ShapeDtypeStruct((_N, _D), jnp.float32)
)

_pre = pl.pallas_call(
    _pre_body,
    out_shape=(
        jax.ShapeDtypeStruct((_N, _D), jnp.float32),
        jax.ShapeDtypeStruct((2 * _N, _D), jnp.float32),
    ),
)

_post = pl.pallas_call(
    _post_body, out_shape=jax.ShapeDtypeStruct((_N, _D), jnp.float32)
)

_head = pl.pallas_call(
    _head_body, out_shape=jax.ShapeDtypeStruct((_N, _D), jnp.float32)
)


_sc_mesh = plsc.VectorSubcoreMesh(
    core_axis_name="c", subcore_axis_name="s", num_cores=_NCORE,
    num_subcores=_NSUB,
)


@functools.partial(
    pl.kernel,
    out_type=jax.ShapeDtypeStruct((2 * _N, _D), jnp.float32),
    mesh=_sc_mesh,
    scratch_types=[
        pltpu.VMEM((_C,), jnp.int32),
        pltpu.VMEM((_C,), jnp.int32),
        pltpu.VMEM((_C, _D), jnp.float32),
        pltpu.VMEM((_ZR, _D), jnp.float32),
        pltpu.VMEM_SHARED((_N, _D), jnp.float32),
        pltpu.SemaphoreType.DMA,
    ],
)
def _sc_segment(tab, srca, dsta, out, srcv, dstv, rows, zbuf, accum, sem):
    c = lax.axis_index("c")
    s = lax.axis_index("s")

    def _zrow(i, carry):
        for j in range(_D // 16):
            zbuf[i, pl.ds(j * 16, 16)] = jnp.zeros((16,), jnp.float32)
        return carry

    lax.fori_loop(0, _ZR, _zrow, 0)
    for k in range(_RPS // _ZR):
        pltpu.sync_copy(zbuf, accum.at[pl.ds(s * _RPS + k * _ZR, _ZR)])

    @pl.when(s == 0)
    def _zero_rem():
        pltpu.sync_copy(
            zbuf.at[pl.ds(0, _REM)], accum.at[pl.ds(_RPS * _NSUB, _REM)]
        )

    plsc.subcore_barrier()

    coff = c * _N

    def _step(j, carry):
        ch = s + j * _NSUB
        off = ch * _C
        pltpu.sync_copy(srca.at[pl.ds(off, _C)], srcv)
        pltpu.sync_copy(dsta.at[pl.ds(off, _C)], dstv)
        for i in range(_C // 16):
            srcv[pl.ds(i * 16, 16)] = srcv[pl.ds(i * 16, 16)] + coff
        pltpu.async_copy(tab.at[srcv], rows, sem).wait()
        pltpu.sync_copy(rows, accum.at[dstv], add=True)
        return carry

    nch = 156 + jnp.where(s < _NCH - 156 * _NSUB, 1, 0)
    lax.fori_loop(0, nch, _step, 0)
    plsc.subcore_barrier()

    pltpu.sync_copy(
        accum.at[pl.ds(s * _RPS, _RPS)],
        out.at[pl.ds(coff + s * _RPS, _RPS)],
    )

    @pl.when(s == 0)
    def _write_rem():
        pltpu.sync_copy(
            accum.at[pl.ds(_RPS * _NSUB, _REM)],
            out.at[pl.ds(coff + _RPS * _NSUB, _REM)],
        )


def kernel(x, edge_index, enc_W, enc_b, ln_w, ln_b, t, W1, b1, mlp_ln_w,
           mlp_ln_b, W2, b2, lin_W, lin_b, out_W, out_b):
    src = edge_index[0]
    dst = edge_index[1]
    h = _encode(x, enc_W, enc_b.reshape(1, _D))
    for i in range(_L):
        z, tab = _pre(
            h,
            ln_w[i].reshape(1, _D),
            ln_b[i].reshape(1, _D),
            t[i].reshape(1, 1),
        )
        s12 = _sc_segment(tab, src, dst)
        h = _post(
            h, z, s12,
            W1[i], b1[i].reshape(1, _H),
            mlp_ln_w[i].reshape(1, _H), mlp_ln_b[i].reshape(1, _H),
            W2[i], b2[i].reshape(1, _D),
        )
    return _head(
        h, lin_W, lin_b.reshape(1, _D), out_W, out_b.reshape(1, _D)
    )


# speedup vs baseline: 13.3800x; 1.4600x over previous
import functools

import jax
import jax.numpy as jnp
from jax import lax
from jax.experimental import pallas as pl
from jax.experimental.pallas import tpu as pltpu
from jax.experimental.pallas import tpu_sc as plsc

_N = 10000
_E = 320000
_D = 128
_H = 256
_L = 3

_C = 128
_NCH = _E // _C
_NSUB = 16
_NCORE = 2
_ZR = 48
_RPS = 624
_REM = _N - _RPS * _NSUB


def _graph_ln(h, w, b, eps=1e-5):
    mu = jnp.mean(h)
    var = jnp.mean((h - mu) ** 2)
    return (h - mu) / (jnp.sqrt(var) + eps) * w + b


def _encode_body(x_ref, w_ref, b_ref, o_ref):
    o_ref[...] = (
        jnp.dot(x_ref[...], w_ref[...], preferred_element_type=jnp.float32)
        + b_ref[...]
    )


def _pre_body(h_ref, lnw_ref, lnb_ref, t_ref, z_ref, tab_ref):
    h = h_ref[...]
    z = jnp.maximum(_graph_ln(h, lnw_ref[...], lnb_ref[...]), 0.0)
    z_ref[...] = z
    msg = z + 1e-7
    a = jnp.exp(msg * t_ref[0, 0])
    tab_ref[0:_N, :] = a
    tab_ref[_N : 2 * _N, :] = msg * a


def _post_body(h_ref, z_ref, s_ref, w1_ref, b1_ref, lnw_ref, lnb_ref,
               w2_ref, b2_ref, o_ref):
    s1 = s_ref[0:_N, :]
    s2 = s_ref[_N : 2 * _N, :]
    out = s2 / (s1 + 1e-16) + z_ref[...]
    h1 = (
        jnp.dot(out, w1_ref[...], preferred_element_type=jnp.float32)
        + b1_ref[...]
    )
    g = jnp.maximum(_graph_ln(h1, lnw_ref[...], lnb_ref[...]), 0.0)
    o_ref[...] = (
        h_ref[...]
        + jnp.dot(g, w2_ref[...], preferred_element_type=jnp.float32)
        + b2_ref[...]
    )


def _head_body(h_ref, lw_ref, lb_ref, ow_ref, ob_ref, o_ref):
    g = jnp.maximum(
        jnp.dot(h_ref[...], lw_ref[...], preferred_element_type=jnp.float32)
        + lb_ref[...],
        0.0,
    )
    o_ref[...] = (
        jnp.dot(g, ow_ref[...], preferred_element_type=jnp.float32)
        + ob_ref[...]
    )


_encode = pl.pallas_call(
    _encode_body, out_shape=jax.ShapeDtypeStruct((_N, _D), jnp.float32)
)

_pre = pl.pallas_call(
    _pre_body,
    out_shape=(
        jax.ShapeDtypeStruct((_N, _D), jnp.float32),
        jax.ShapeDtypeStruct((2 * _N, _D), jnp.float32),
    ),
)

_post = pl.pallas_call(
    _post_body, out_shape=jax.ShapeDtypeStruct((_N, _D), jnp.float32)
)

_head = pl.pallas_call(
    _head_body, out_shape=jax.ShapeDtypeStruct((_N, _D), jnp.float32)
)


_sc_mesh = plsc.VectorSubcoreMesh(
    core_axis_name="c", subcore_axis_name="s", num_cores=_NCORE,
    num_subcores=_NSUB,
)


_RING = 2
_NBLK = 2496 // _NSUB // _RING


@functools.partial(
    pl.kernel,
    out_type=jax.ShapeDtypeStruct((2 * _N, _D), jnp.float32),
    mesh=_sc_mesh,
    scratch_types=[
        pltpu.VMEM((_RING, _C), jnp.int32),
        pltpu.VMEM((_RING, _C), jnp.int32),
        pltpu.VMEM((_RING, _C, _D), jnp.float32),
        pltpu.VMEM((_ZR, _D), jnp.float32),
        pltpu.VMEM_SHARED((_N, _D), jnp.float32),
        pltpu.SemaphoreType.DMA((_RING,)),
        pltpu.SemaphoreType.DMA((_RING,)),
        pltpu.SemaphoreType.DMA((_RING,)),
        pltpu.SemaphoreType.DMA((_RING,)),
    ],
)
def _sc_segment(tab, srca, dsta, out, srcv, dstv, rows, zbuf, accum,
                ssem, dsem, gsem, csem):
    c = lax.axis_index("c")
    s = lax.axis_index("s")

    def _zrow(i, carry):
        for j in range(_D // 16):
            zbuf[i, pl.ds(j * 16, 16)] = jnp.zeros((16,), jnp.float32)
        return carry

    lax.fori_loop(0, _ZR, _zrow, 0)
    for k in range(_RPS // _ZR):
        pltpu.sync_copy(zbuf, accum.at[pl.ds(s * _RPS + k * _ZR, _ZR)])

    @pl.when(s == 0)
    def _zero_rem():
        pltpu.sync_copy(
            zbuf.at[pl.ds(0, _REM)], accum.at[pl.ds(_RPS * _NSUB, _REM)]
        )

    plsc.subcore_barrier()

    coff = c * _N

    def _block(jb, carry):
        sds, dds = [], []
        for k in range(_RING):
            off = (s + (jb * _RING + k) * _NSUB) * _C
            sds.append(
                pltpu.async_copy(srca.at[pl.ds(off, _C)], srcv.at[k],
                                 ssem.at[k])
            )
            dds.append(
                pltpu.async_copy(dsta.at[pl.ds(off, _C)], dstv.at[k],
                                 dsem.at[k])
            )
        gds = []
        for k in range(_RING):
            sds[k].wait()
            for i in range(_C // 16):
                srcv[k, pl.ds(i * 16, 16)] = srcv[k, pl.ds(i * 16, 16)] + coff
            gds.append(
                pltpu.async_copy(tab.at[srcv.at[k]], rows.at[k], gsem.at[k])
            )
        cds = []
        for k in range(_RING):
            dds[k].wait()
            gds[k].wait()
            cds.append(
                pltpu.async_copy(rows.at[k], accum.at[dstv.at[k]],
                                 csem.at[k], add=True)
            )
        for d in cds:
            d.wait()
        return carry

    lax.fori_loop(0, _NBLK, _block, 0)

    @pl.when(s < _NCH - _NBLK * _RING * _NSUB)
    def _tail():
        off = (_NBLK * _RING * _NSUB + s) * _C
        pltpu.async_copy(srca.at[pl.ds(off, _C)], srcv.at[0], ssem.at[0]).wait()
        pltpu.async_copy(dsta.at[pl.ds(off, _C)], dstv.at[0], dsem.at[0]).wait()
        for i in range(_C // 16):
            srcv[0, pl.ds(i * 16, 16)] = srcv[0, pl.ds(i * 16, 16)] + coff
        pltpu.async_copy(tab.at[srcv.at[0]], rows.at[0], gsem.at[0]).wait()
        pltpu.async_copy(
            rows.at[0], accum.at[dstv.at[0]], csem.at[0], add=True
        ).wait()

    plsc.subcore_barrier()

    pltpu.sync_copy(
        accum.at[pl.ds(s * _RPS, _RPS)],
        out.at[pl.ds(coff + s * _RPS, _RPS)],
    )

    @pl.when(s == 0)
    def _write_rem():
        pltpu.sync_copy(
            accum.at[pl.ds(_RPS * _NSUB, _REM)],
            out.at[pl.ds(coff + _RPS * _NSUB, _REM)],
        )


def kernel(x, edge_index, enc_W, enc_b, ln_w, ln_b, t, W1, b1, mlp_ln_w,
           mlp_ln_b, W2, b2, lin_W, lin_b, out_W, out_b):
    src = edge_index[0]
    dst = edge_index[1]
    h = _encode(x, enc_W, enc_b.reshape(1, _D))
    for i in range(_L):
        z, tab = _pre(
            h,
            ln_w[i].reshape(1, _D),
            ln_b[i].reshape(1, _D),
            t[i].reshape(1, 1),
        )
        s12 = _sc_segment(tab, src, dst)
        h = _post(
            h, z, s12,
            W1[i], b1[i].reshape(1, _H),
            mlp_ln_w[i].reshape(1, _H), mlp_ln_b[i].reshape(1, _H),
            W2[i], b2[i].reshape(1, _D),
        )
    return _head(
        h, lin_W, lin_b.reshape(1, _D), out_W, out_b.reshape(1, _D)
    )


# speedup vs baseline: 16.5429x; 1.2364x over previous
import functools

import jax
import jax.numpy as jnp
from jax import lax
from jax.experimental import pallas as pl
from jax.experimental.pallas import tpu as pltpu
from jax.experimental.pallas import tpu_sc as plsc

_N = 10000
_E = 320000
_D = 128
_H = 256
_L = 3

_C = 128
_NCH = _E // _C
_NSUB = 16
_NCORE = 2
_ZR = 48
_RPS = 624
_REM = _N - _RPS * _NSUB


def _graph_ln(h, w, b, eps=1e-5):
    mu = jnp.mean(h)
    var = jnp.mean((h - mu) ** 2)
    return (h - mu) / (jnp.sqrt(var) + eps) * w + b


def _encode_body(x_ref, w_ref, b_ref, o_ref):
    o_ref[...] = (
        jnp.dot(x_ref[...], w_ref[...], preferred_element_type=jnp.float32)
        + b_ref[...]
    )


def _pre_body(h_ref, lnw_ref, lnb_ref, t_ref, z_ref, tab_ref):
    h = h_ref[...]
    z = jnp.maximum(_graph_ln(h, lnw_ref[...], lnb_ref[...]), 0.0)
    z_ref[...] = z
    msg = z + 1e-7
    a = jnp.exp(msg * t_ref[0, 0])
    tab_ref[0:_N, :] = a
    tab_ref[_N : 2 * _N, :] = msg * a


def _post_body(h_ref, z_ref, s_ref, w1_ref, b1_ref, lnw_ref, lnb_ref,
               w2_ref, b2_ref, o_ref):
    s1 = s_ref[0:_N, :]
    s2 = s_ref[_N : 2 * _N, :]
    out = s2 / (s1 + 1e-16) + z_ref[...]
    h1 = (
        jnp.dot(out, w1_ref[...], preferred_element_type=jnp.float32)
        + b1_ref[...]
    )
    g = jnp.maximum(_graph_ln(h1, lnw_ref[...], lnb_ref[...]), 0.0)
    o_ref[...] = (
        h_ref[...]
        + jnp.dot(g, w2_ref[...], preferred_element_type=jnp.float32)
        + b2_ref[...]
    )


def _head_body(h_ref, lw_ref, lb_ref, ow_ref, ob_ref, o_ref):
    g = jnp.maximum(
        jnp.dot(h_ref[...], lw_ref[...], preferred_element_type=jnp.float32)
        + lb_ref[...],
        0.0,
    )
    o_ref[...] = (
        jnp.dot(g, ow_ref[...], preferred_element_type=jnp.float32)
        + ob_ref[...]
    )


_encode = pl.pallas_call(
    _encode_body, out_shape=jax.ShapeDtypeStruct((_N, _D), jnp.float32)
)

_pre = pl.pallas_call(
    _pre_body,
    out_shape=(
        jax.ShapeDtypeStruct((_N, _D), jnp.float32),
        jax.ShapeDtypeStruct((2 * _N, _D), jnp.float32),
    ),
)

_post = pl.pallas_call(
    _post_body, out_shape=jax.ShapeDtypeStruct((_N, _D), jnp.float32)
)

_head = pl.pallas_call(
    _head_body, out_shape=jax.ShapeDtypeStruct((_N, _D), jnp.float32)
)


_sc_mesh = plsc.VectorSubcoreMesh(
    core_axis_name="c", subcore_axis_name="s", num_cores=_NCORE,
    num_subcores=_NSUB,
)


_U = 6
_CPS = 2496 // _NSUB
_NBLK = _CPS // _U


@functools.partial(
    pl.kernel,
    out_type=jax.ShapeDtypeStruct((2 * _N, _D), jnp.float32),
    mesh=_sc_mesh,
    scratch_types=[
        pltpu.VMEM((_U * _C,), jnp.int32),
        pltpu.VMEM((_U, _C), jnp.int32),
        pltpu.VMEM((2, _C, _D), jnp.float32),
        pltpu.VMEM((_ZR, _D), jnp.float32),
        pltpu.VMEM_SHARED((_N, _D), jnp.float32),
        pltpu.SemaphoreType.DMA,
        pltpu.SemaphoreType.DMA((_U,)),
        pltpu.SemaphoreType.DMA((2,)),
        pltpu.SemaphoreType.DMA((2,)),
    ],
)
def _sc_segment(tab, srca, dsta, out, srcb, dstb, rows, zbuf, accum,
                ssem, dsem, gsem, csem):
    c = lax.axis_index("c")
    s = lax.axis_index("s")

    def _zrow(i, carry):
        for j in range(_D // 16):
            zbuf[i, pl.ds(j * 16, 16)] = jnp.zeros((16,), jnp.float32)
        return carry

    lax.fori_loop(0, _ZR, _zrow, 0)
    for k in range(_RPS // _ZR):
        pltpu.sync_copy(zbuf, accum.at[pl.ds(s * _RPS + k * _ZR, _ZR)])

    @pl.when(s == 0)
    def _zero_rem():
        pltpu.sync_copy(
            zbuf.at[pl.ds(0, _REM)], accum.at[pl.ds(_RPS * _NSUB, _REM)]
        )

    plsc.subcore_barrier()

    def _block(jb, carry):
        off0 = (s * _CPS + jb * _U) * _C
        sd = pltpu.async_copy(
            srca.at[pl.ds(c * _E + off0, _U * _C)], srcb, ssem
        )
        dds = [
            pltpu.async_copy(dsta.at[pl.ds(off0 + u * _C, _C)], dstb.at[u],
                             dsem.at[u])
            for u in range(_U)
        ]
        sd.wait()
        g = [None] * _U
        cds = [None] * _U
        g[0] = pltpu.async_copy(tab.at[srcb.at[pl.ds(0, _C)]], rows.at[0],
                                gsem.at[0])
        g[1] = pltpu.async_copy(tab.at[srcb.at[pl.ds(_C, _C)]], rows.at[1],
                                gsem.at[1])
        for u in range(_U):
            g[u].wait()
            dds[u].wait()
            cds[u] = pltpu.async_copy(rows.at[u % 2], accum.at[dstb.at[u]],
                                      csem.at[u % 2], add=True)
            if u + 2 < _U:
                cds[u].wait()
                g[u + 2] = pltpu.async_copy(
                    tab.at[srcb.at[pl.ds((u + 2) * _C, _C)]],
                    rows.at[u % 2], gsem.at[u % 2],
                )
        cds[_U - 2].wait()
        cds[_U - 1].wait()
        return carry

    lax.fori_loop(0, _NBLK, _block, 0)

    @pl.when(s < _NCH - _CPS * _NSUB)
    def _tail():
        off = (_CPS * _NSUB + s) * _C
        pltpu.async_copy(
            srca.at[pl.ds(c * _E + off, _C)], srcb.at[pl.ds(0, _C)], ssem
        ).wait()
        pltpu.async_copy(dsta.at[pl.ds(off, _C)], dstb.at[0], dsem.at[0]).wait()
        pltpu.async_copy(
            tab.at[srcb.at[pl.ds(0, _C)]], rows.at[0], gsem.at[0]
        ).wait()
        pltpu.async_copy(
            rows.at[0], accum.at[dstb.at[0]], csem.at[0], add=True
        ).wait()

    plsc.subcore_barrier()

    coff = c * _N
    pltpu.sync_copy(
        accum.at[pl.ds(s * _RPS, _RPS)],
        out.at[pl.ds(coff + s * _RPS, _RPS)],
    )

    @pl.when(s == 0)
    def _write_rem():
        pltpu.sync_copy(
            accum.at[pl.ds(_RPS * _NSUB, _REM)],
            out.at[pl.ds(coff + _RPS * _NSUB, _REM)],
        )


def kernel(x, edge_index, enc_W, enc_b, ln_w, ln_b, t, W1, b1, mlp_ln_w,
           mlp_ln_b, W2, b2, lin_W, lin_b, out_W, out_b):
    src = edge_index[0]
    dst = edge_index[1]
    srcsh = jnp.concatenate([src, src + _N])
    h = _encode(x, enc_W, enc_b.reshape(1, _D))
    for i in range(_L):
        z, tab = _pre(
            h,
            ln_w[i].reshape(1, _D),
            ln_b[i].reshape(1, _D),
            t[i].reshape(1, 1),
        )
        s12 = _sc_segment(tab, srcsh, dst)
        h = _post(
            h, z, s12,
            W1[i], b1[i].reshape(1, _H),
            mlp_ln_w[i].reshape(1, _H), mlp_ln_b[i].reshape(1, _H),
            W2[i], b2[i].reshape(1, _D),
        )
    return _head(
        h, lin_W, lin_b.reshape(1, _D), out_W, out_b.reshape(1, _D)
    )
